# baseline (speedup 1.0000x reference)
import functools

import jax
import jax.numpy as jnp
from jax import lax
from jax.experimental import pallas as pl
from jax.experimental.pallas import tpu as pltpu
from jax.experimental.pallas import tpu_sc as plsc

N_NODES = 10000
N_PAD = 10240
E_EDGES = 320000
GRP = 128
NGRP = 80
NWORK = 32
E_PAD = NWORK * NGRP * GRP
ROWS_PER_TILE = N_PAD // 16
H = 64
DEGW = 16
GSZ = 4
NDMA = NGRP // GSZ

_sc_mesh = plsc.VectorSubcoreMesh(core_axis_name="c", subcore_axis_name="s")
_sc_params = pltpu.CompilerParams(use_tc_tiling_on_sc=False)


def _deg_body(dst_hbm, ones_hbm, zero_hbm, out_hbm, idx_v, ones_v, acc):
    c = lax.axis_index("c")
    s = lax.axis_index("s")
    wid = c * 16 + s
    rows = pl.ds(s * ROWS_PER_TILE, ROWS_PER_TILE)
    pltpu.sync_copy(zero_hbm.at[rows], acc.at[rows])
    pltpu.sync_copy(ones_hbm, ones_v)
    pltpu.sync_copy(dst_hbm.at[wid], idx_v)
    plsc.subcore_barrier()

    def body(j, carry):
        pltpu.sync_copy(ones_v, acc.at[idx_v.at[j]], add=True)
        return carry

    lax.fori_loop(0, NGRP, body, 0)
    plsc.subcore_barrier()
    pltpu.sync_copy(acc.at[rows], out_hbm.at[c, rows])


_deg_kernel = functools.partial(
    pl.kernel,
    out_type=jax.ShapeDtypeStruct((2, N_PAD, DEGW), jnp.float32),
    mesh=_sc_mesh,
    scratch_types=[
        pltpu.VMEM((NGRP, GRP), jnp.int32),
        pltpu.VMEM((GRP, DEGW), jnp.float32),
        pltpu.VMEM_SHARED((N_PAD, DEGW), jnp.float32),
    ],
    compiler_params=_sc_params,
)(_deg_body)


def _agg_body(u_hbm, src_hbm, dst_hbm, out_hbm,
              idx_s, idx_d, rows_v, zbuf, acc, gsem, ssem):
    c = lax.axis_index("c")
    s = lax.axis_index("s")
    wid = c * 16 + s
    rows = pl.ds(s * ROWS_PER_TILE, ROWS_PER_TILE)
    for i in range(64):
        for k in range(4):
            zbuf[i, pl.ds(k * 16, 16)] = jnp.zeros((16,), jnp.float32)
    zdescs = [
        pltpu.async_copy(
            zbuf, acc.at[pl.ds(s * ROWS_PER_TILE + r * 64, 64)], ssem)
        for r in range(ROWS_PER_TILE // 64)
    ]
    for d in zdescs:
        d.wait()
    pltpu.sync_copy(src_hbm.at[wid], idx_s)
    pltpu.sync_copy(dst_hbm.at[wid], idx_d)
    plsc.subcore_barrier()

    def body(g, carry):
        pltpu.async_copy(u_hbm.at[idx_s.at[g]], rows_v, gsem).wait()
        pltpu.sync_copy(rows_v, acc.at[idx_d.at[g]], add=True)
        return carry

    lax.fori_loop(0, NGRP, body, 0)

    plsc.subcore_barrier()
    pltpu.sync_copy(acc.at[rows], out_hbm.at[c, rows])


_agg_kernel = functools.partial(
    pl.kernel,
    out_type=jax.ShapeDtypeStruct((2, N_PAD, H), jnp.float32),
    mesh=_sc_mesh,
    scratch_types=[
        pltpu.VMEM((NGRP, GRP), jnp.int32),
        pltpu.VMEM((NGRP, GRP), jnp.int32),
        pltpu.VMEM((GRP, H), jnp.float32),
        pltpu.VMEM((64, H), jnp.float32),
        pltpu.VMEM_SHARED((N_PAD, H), jnp.float32),
        pltpu.SemaphoreType.DMA,
        pltpu.SemaphoreType.DMA,
    ],
    compiler_params=_sc_params,
)(_agg_body)


def _dinv_of(degp_blk):
    deg = degp_blk[0, :, 0] + degp_blk[1, :, 0] + 1.0
    return lax.rsqrt(jnp.maximum(deg, 1.0))


def _tc_first_body(x_ref, degp_ref, w_in_ref, b_in_ref, w1_ref, u_ref):
    dinv = _dinv_of(degp_ref[...])
    h = jnp.dot(x_ref[...], w_in_ref[...], preferred_element_type=jnp.float32)
    h = jax.nn.relu(h + b_in_ref[...][None, :])
    u = jnp.dot(h, w1_ref[...], preferred_element_type=jnp.float32)
    u_ref[...] = u * dinv[:, None]


def _layer_norm(t, g, b, eps=1e-5):
    mu = jnp.mean(t, axis=-1, keepdims=True)
    var = jnp.mean((t - mu) ** 2, axis=-1, keepdims=True)
    return (t - mu) * lax.rsqrt(var + eps) * g[None, :] + b[None, :]


def _tc_mid_body(p_ref, u_ref, degp_ref, b_ref, g_ref, be_ref, w_ref,
                 un_ref):
    dinv = _dinv_of(degp_ref[...])
    p = p_ref[...]
    agg = p[0] + p[1] + u_ref[...]
    t = agg * dinv[:, None] + b_ref[...][None, :]
    t = _layer_norm(jax.nn.relu(t), g_ref[...], be_ref[...])
    un = jnp.dot(t, w_ref[...], preferred_element_type=jnp.float32)
    un_ref[...] = un * dinv[:, None]


def _tc_last_body(p_ref, u_ref, degp_ref, b_ref, g_ref, be_ref, w_out_ref,
                  b_out_ref, out_ref):
    dinv = _dinv_of(degp_ref[...])
    p = p_ref[...]
    agg = p[0] + p[1] + u_ref[...]
    t = agg * dinv[:, None] + b_ref[...][None, :]
    t = _layer_norm(jax.nn.relu(t), g_ref[...], be_ref[...])
    out = jnp.dot(t, w_out_ref[...], preferred_element_type=jnp.float32)
    out_ref[...] = out + b_out_ref[...][None, :]


_TCB = 1024
_GRID = (N_PAD // _TCB,)


def _rowblk(width):
    return pl.BlockSpec((_TCB, width), lambda i: (i, 0))


def _degblk():
    return pl.BlockSpec((2, _TCB, DEGW), lambda i: (0, i, 0))


def _pblk():
    return pl.BlockSpec((2, _TCB, H), lambda i: (0, i, 0))


def _full2(a, b):
    return pl.BlockSpec((a, b), lambda i: (0, 0))


def _full1(a):
    return pl.BlockSpec((a,), lambda i: (0,))


def _tc_first(x_pad, degp, w_in, b_in, w1):
    return pl.pallas_call(
        _tc_first_body,
        grid=_GRID,
        in_specs=[_rowblk(128), _degblk(), _full2(128, H), _full1(H),
                  _full2(H, H)],
        out_specs=_rowblk(H),
        out_shape=jax.ShapeDtypeStruct((N_PAD, H), jnp.float32),
    )(x_pad, degp, w_in, b_in, w1)


def _tc_mid(p, u, degp, b, g, be, w_next):
    return pl.pallas_call(
        _tc_mid_body,
        grid=_GRID,
        in_specs=[_pblk(), _rowblk(H), _degblk(), _full1(H), _full1(H),
                  _full1(H), _full2(H, H)],
        out_specs=_rowblk(H),
        out_shape=jax.ShapeDtypeStruct((N_PAD, H), jnp.float32),
    )(p, u, degp, b, g, be, w_next)


def _tc_last(p, u, degp, b, g, be, w_out, b_out):
    return pl.pallas_call(
        _tc_last_body,
        grid=_GRID,
        in_specs=[_pblk(), _rowblk(H), _degblk(), _full1(H), _full1(H),
                  _full1(H), _full2(H, 40), _full1(40)],
        out_specs=_rowblk(40),
        out_shape=jax.ShapeDtypeStruct((N_PAD, 40), jnp.float32),
    )(p, u, degp, b, g, be, w_out, b_out)


def kernel(x, edge_index, batch, W_in, b_in, W1, b1, g1, be1, W2, b2, g2,
           be2, W3, b3, g3, be3, W_out, b_out):
    x_pad = jnp.zeros((N_PAD, 128), jnp.float32).at[:N_NODES].set(x)
    pad_idx = jnp.full((E_PAD - E_EDGES,), N_NODES, jnp.int32)
    src = jnp.concatenate([edge_index[0], pad_idx]).reshape(NWORK, NGRP, GRP)
    dst = jnp.concatenate([edge_index[1], pad_idx]).reshape(NWORK, NGRP, GRP)

    zeros_d = jnp.zeros((N_PAD, DEGW), jnp.float32)
    ones_d = jnp.ones((GRP, DEGW), jnp.float32)

    degp = _deg_kernel(dst, ones_d, zeros_d)
    u1 = _tc_first(x_pad, degp, W_in, b_in, W1)
    p1 = _agg_kernel(u1, src, dst)
    u2 = _tc_mid(p1, u1, degp, b1, g1, be1, W2)
    p2 = _agg_kernel(u2, src, dst)
    u3 = _tc_mid(p2, u2, degp, b2, g2, be2, W3)
    p3 = _agg_kernel(u3, src, dst)
    out = _tc_last(p3, u3, degp, b3, g3, be3, W_out, b_out)
    return out[:N_NODES]


# speedup vs baseline: 1.0251x; 1.0251x over previous
import functools

import jax
import jax.numpy as jnp
from jax import lax
from jax.experimental import pallas as pl
from jax.experimental.pallas import tpu as pltpu
from jax.experimental.pallas import tpu_sc as plsc

N_NODES = 10000
N_PAD = 10240
E_EDGES = 320000
GRP = 128
NGRP = 80
NWORK = 32
E_PAD = NWORK * NGRP * GRP
ROWS_PER_TILE = N_PAD // 16
H = 64
DEGW = 16
GSZ = 4
NDMA = NGRP // GSZ

_sc_mesh = plsc.VectorSubcoreMesh(core_axis_name="c", subcore_axis_name="s")
_sc_params = pltpu.CompilerParams(use_tc_tiling_on_sc=False)


def _deg_body(dst_hbm, ones_hbm, zero_hbm, out_hbm, idx_v, ones_v, acc):
    c = lax.axis_index("c")
    s = lax.axis_index("s")
    wid = c * 16 + s
    rows = pl.ds(s * ROWS_PER_TILE, ROWS_PER_TILE)
    pltpu.sync_copy(zero_hbm.at[rows], acc.at[rows])
    pltpu.sync_copy(ones_hbm, ones_v)
    pltpu.sync_copy(dst_hbm.at[wid], idx_v)
    plsc.subcore_barrier()

    def body(j, carry):
        pltpu.sync_copy(ones_v, acc.at[idx_v.at[j]], add=True)
        return carry

    lax.fori_loop(0, NGRP, body, 0)
    plsc.subcore_barrier()
    pltpu.sync_copy(acc.at[rows], out_hbm.at[c, rows])


_deg_kernel = functools.partial(
    pl.kernel,
    out_type=jax.ShapeDtypeStruct((2, N_PAD, DEGW), jnp.float32),
    mesh=_sc_mesh,
    scratch_types=[
        pltpu.VMEM((NGRP, GRP), jnp.int32),
        pltpu.VMEM((GRP, DEGW), jnp.float32),
        pltpu.VMEM_SHARED((N_PAD, DEGW), jnp.float32),
    ],
    compiler_params=_sc_params,
)(_deg_body)


def _agg_body(u_hbm, src_hbm, dst_hbm, zero_hbm, out_hbm,
              idx_s, idx_d, rows_v, acc, gsem, ssem):
    c = lax.axis_index("c")
    s = lax.axis_index("s")
    wid = c * 16 + s
    rows = pl.ds(s * ROWS_PER_TILE, ROWS_PER_TILE)
    pltpu.sync_copy(zero_hbm.at[rows], acc.at[rows])
    pltpu.sync_copy(src_hbm.at[wid], idx_s)
    pltpu.sync_copy(dst_hbm.at[wid], idx_d)
    plsc.subcore_barrier()

    def body(g, carry):
        pltpu.async_copy(u_hbm.at[idx_s.at[g]], rows_v, gsem).wait()
        pltpu.sync_copy(rows_v, acc.at[idx_d.at[g]], add=True)
        return carry

    lax.fori_loop(0, NGRP, body, 0)

    plsc.subcore_barrier()
    pltpu.sync_copy(acc.at[rows], out_hbm.at[c, rows])


_agg_kernel = functools.partial(
    pl.kernel,
    out_type=jax.ShapeDtypeStruct((2, N_PAD, H), jnp.float32),
    mesh=_sc_mesh,
    scratch_types=[
        pltpu.VMEM((NGRP, GRP), jnp.int32),
        pltpu.VMEM((NGRP, GRP), jnp.int32),
        pltpu.VMEM((GRP, H), jnp.float32),
        pltpu.VMEM_SHARED((N_PAD, H), jnp.float32),
        pltpu.SemaphoreType.DMA,
        pltpu.SemaphoreType.DMA,
    ],
    compiler_params=_sc_params,
)(_agg_body)


def _dinv_of(degp_blk):
    deg = degp_blk[0, :, 0] + degp_blk[1, :, 0] + 1.0
    return lax.rsqrt(jnp.maximum(deg, 1.0))


def _tc_first_body(x_ref, degp_ref, w_in_ref, b_in_ref, w1_ref, u_ref):
    dinv = _dinv_of(degp_ref[...])
    h = jnp.dot(x_ref[...], w_in_ref[...], preferred_element_type=jnp.float32)
    h = jax.nn.relu(h + b_in_ref[...][None, :])
    u = jnp.dot(h, w1_ref[...], preferred_element_type=jnp.float32)
    u_ref[...] = u * dinv[:, None]


def _layer_norm(t, g, b, eps=1e-5):
    mu = jnp.mean(t, axis=-1, keepdims=True)
    var = jnp.mean((t - mu) ** 2, axis=-1, keepdims=True)
    return (t - mu) * lax.rsqrt(var + eps) * g[None, :] + b[None, :]


def _tc_mid_body(p_ref, u_ref, degp_ref, b_ref, g_ref, be_ref, w_ref,
                 un_ref):
    dinv = _dinv_of(degp_ref[...])
    p = p_ref[...]
    agg = p[0] + p[1] + u_ref[...]
    t = agg * dinv[:, None] + b_ref[...][None, :]
    t = _layer_norm(jax.nn.relu(t), g_ref[...], be_ref[...])
    un = jnp.dot(t, w_ref[...], preferred_element_type=jnp.float32)
    un_ref[...] = un * dinv[:, None]


def _tc_last_body(p_ref, u_ref, degp_ref, b_ref, g_ref, be_ref, w_out_ref,
                  b_out_ref, out_ref):
    dinv = _dinv_of(degp_ref[...])
    p = p_ref[...]
    agg = p[0] + p[1] + u_ref[...]
    t = agg * dinv[:, None] + b_ref[...][None, :]
    t = _layer_norm(jax.nn.relu(t), g_ref[...], be_ref[...])
    out = jnp.dot(t, w_out_ref[...], preferred_element_type=jnp.float32)
    out_ref[...] = out + b_out_ref[...][None, :]


_TCB = 1024
_GRID = (N_PAD // _TCB,)


def _rowblk(width):
    return pl.BlockSpec((_TCB, width), lambda i: (i, 0))


def _degblk():
    return pl.BlockSpec((2, _TCB, DEGW), lambda i: (0, i, 0))


def _pblk():
    return pl.BlockSpec((2, _TCB, H), lambda i: (0, i, 0))


def _full2(a, b):
    return pl.BlockSpec((a, b), lambda i: (0, 0))


def _full1(a):
    return pl.BlockSpec((a,), lambda i: (0,))


def _tc_first(x_pad, degp, w_in, b_in, w1):
    return pl.pallas_call(
        _tc_first_body,
        grid=_GRID,
        in_specs=[_rowblk(128), _degblk(), _full2(128, H), _full1(H),
                  _full2(H, H)],
        out_specs=_rowblk(H),
        out_shape=jax.ShapeDtypeStruct((N_PAD, H), jnp.float32),
    )(x_pad, degp, w_in, b_in, w1)


def _tc_mid(p, u, degp, b, g, be, w_next):
    return pl.pallas_call(
        _tc_mid_body,
        grid=_GRID,
        in_specs=[_pblk(), _rowblk(H), _degblk(), _full1(H), _full1(H),
                  _full1(H), _full2(H, H)],
        out_specs=_rowblk(H),
        out_shape=jax.ShapeDtypeStruct((N_PAD, H), jnp.float32),
    )(p, u, degp, b, g, be, w_next)


def _tc_last(p, u, degp, b, g, be, w_out, b_out):
    return pl.pallas_call(
        _tc_last_body,
        grid=_GRID,
        in_specs=[_pblk(), _rowblk(H), _degblk(), _full1(H), _full1(H),
                  _full1(H), _full2(H, 40), _full1(40)],
        out_specs=_rowblk(40),
        out_shape=jax.ShapeDtypeStruct((N_PAD, 40), jnp.float32),
    )(p, u, degp, b, g, be, w_out, b_out)


def kernel(x, edge_index, batch, W_in, b_in, W1, b1, g1, be1, W2, b2, g2,
           be2, W3, b3, g3, be3, W_out, b_out):
    x_pad = jnp.zeros((N_PAD, 128), jnp.float32).at[:N_NODES].set(x)
    pad_idx = jnp.full((E_PAD - E_EDGES,), N_NODES, jnp.int32)
    src = jnp.concatenate([edge_index[0], pad_idx]).reshape(NWORK, NGRP, GRP)
    dst = jnp.concatenate([edge_index[1], pad_idx]).reshape(NWORK, NGRP, GRP)

    zeros_h = jnp.zeros((N_PAD, H), jnp.float32)
    zeros_d = jnp.zeros((N_PAD, DEGW), jnp.float32)
    ones_d = jnp.ones((GRP, DEGW), jnp.float32)

    degp = _deg_kernel(dst, ones_d, zeros_d)
    u1 = _tc_first(x_pad, degp, W_in, b_in, W1)
    p1 = _agg_kernel(u1, src, dst, zeros_h)
    u2 = _tc_mid(p1, u1, degp, b1, g1, be1, W2)
    p2 = _agg_kernel(u2, src, dst, zeros_h)
    u3 = _tc_mid(p2, u2, degp, b2, g2, be2, W3)
    p3 = _agg_kernel(u3, src, dst, zeros_h)
    out = _tc_last(p3, u3, degp, b3, g3, be3, W_out, b_out)
    return out[:N_NODES]


# speedup vs baseline: 1.8227x; 1.7781x over previous
import functools

import jax
import jax.numpy as jnp
from jax import lax
from jax.experimental import pallas as pl
from jax.experimental.pallas import tpu as pltpu
from jax.experimental.pallas import tpu_sc as plsc

N_NODES = 10000
N_PAD = 10240
E_EDGES = 320000
GRP = 128
NGRP = 79
NWORK = 32
E_PAD = NWORK * NGRP * GRP
ROWS_PER_TILE = N_PAD // 16
H = 64
DEGW = 16
GSZ = 4
NDMA = NGRP // GSZ

_sc_mesh = plsc.VectorSubcoreMesh(core_axis_name="c", subcore_axis_name="s")
_sc_params = pltpu.CompilerParams(use_tc_tiling_on_sc=False)


def _deg_body(dst_hbm, ones_hbm, zero_hbm, out_hbm, idx_v, ones_v, acc):
    c = lax.axis_index("c")
    s = lax.axis_index("s")
    wid = c * 16 + s
    rows = pl.ds(s * ROWS_PER_TILE, ROWS_PER_TILE)
    pltpu.sync_copy(zero_hbm.at[rows], acc.at[rows])
    pltpu.sync_copy(ones_hbm, ones_v)
    pltpu.sync_copy(dst_hbm.at[wid], idx_v)
    plsc.subcore_barrier()

    def body(j, carry):
        pltpu.sync_copy(ones_v, acc.at[idx_v.at[j]], add=True)
        return carry

    lax.fori_loop(0, NGRP, body, 0)
    plsc.subcore_barrier()
    pltpu.sync_copy(acc.at[rows], out_hbm.at[c, rows])


_deg_kernel = functools.partial(
    pl.kernel,
    out_type=jax.ShapeDtypeStruct((2, N_PAD, DEGW), jnp.float32),
    mesh=_sc_mesh,
    scratch_types=[
        pltpu.VMEM((NGRP, GRP), jnp.int32),
        pltpu.VMEM((GRP, DEGW), jnp.float32),
        pltpu.VMEM_SHARED((N_PAD, DEGW), jnp.float32),
    ],
    compiler_params=_sc_params,
)(_deg_body)


def _agg_body(u_hbm, src_hbm, dst_hbm, zero_hbm, out_hbm,
              idx_s, idx_d, rows_v, acc, gsem):
    c = lax.axis_index("c")
    s = lax.axis_index("s")
    wid = c * 16 + s
    rows = pl.ds(s * ROWS_PER_TILE, ROWS_PER_TILE)
    pltpu.sync_copy(zero_hbm.at[rows], acc.at[rows])
    pltpu.sync_copy(src_hbm.at[wid], idx_s)
    pltpu.sync_copy(dst_hbm.at[wid], idx_d)
    plsc.subcore_barrier()

    def body(g, carry):
        pltpu.async_copy(u_hbm.at[idx_s.at[g]], rows_v, gsem).wait()
        pltpu.sync_copy(rows_v, acc.at[idx_d.at[g]], add=True)
        return carry

    lax.fori_loop(0, NGRP, body, 0)

    plsc.subcore_barrier()
    pltpu.sync_copy(acc.at[rows], out_hbm.at[c, rows])


_agg_kernel = functools.partial(
    pl.kernel,
    out_type=jax.ShapeDtypeStruct((2, N_PAD, H), jnp.float32),
    mesh=_sc_mesh,
    scratch_types=[
        pltpu.VMEM((NGRP, GRP), jnp.int32),
        pltpu.VMEM((NGRP, GRP), jnp.int32),
        pltpu.VMEM((GRP, H), jnp.float32),
        pltpu.VMEM_SHARED((N_PAD, H), jnp.float32),
        pltpu.SemaphoreType.DMA,
    ],
    compiler_params=_sc_params,
)(_agg_body)


def _dinv_of(degp_blk):
    deg = degp_blk[0, :, 0] + degp_blk[1, :, 0] + 1.0
    return lax.rsqrt(jnp.maximum(deg, 1.0))


def _tc_first_body(x_ref, degp_ref, w_in_ref, b_in_ref, w1_ref, u_ref):
    dinv = _dinv_of(degp_ref[...])
    h = jnp.dot(x_ref[...], w_in_ref[...], preferred_element_type=jnp.float32)
    h = jax.nn.relu(h + b_in_ref[...][None, :])
    u = jnp.dot(h, w1_ref[...], preferred_element_type=jnp.float32)
    u_ref[...] = u * dinv[:, None]


def _layer_norm(t, g, b, eps=1e-5):
    mu = jnp.mean(t, axis=-1, keepdims=True)
    var = jnp.mean((t - mu) ** 2, axis=-1, keepdims=True)
    return (t - mu) * lax.rsqrt(var + eps) * g[None, :] + b[None, :]


def _tc_mid_body(p_ref, u_ref, degp_ref, b_ref, g_ref, be_ref, w_ref,
                 un_ref):
    dinv = _dinv_of(degp_ref[...])
    p = p_ref[...]
    agg = p[0] + p[1] + u_ref[...]
    t = agg * dinv[:, None] + b_ref[...][None, :]
    t = _layer_norm(jax.nn.relu(t), g_ref[...], be_ref[...])
    un = jnp.dot(t, w_ref[...], preferred_element_type=jnp.float32)
    un_ref[...] = un * dinv[:, None]


def _tc_last_body(p_ref, u_ref, degp_ref, b_ref, g_ref, be_ref, w_out_ref,
                  b_out_ref, out_ref):
    dinv = _dinv_of(degp_ref[...])
    p = p_ref[...]
    agg = p[0] + p[1] + u_ref[...]
    t = agg * dinv[:, None] + b_ref[...][None, :]
    t = _layer_norm(jax.nn.relu(t), g_ref[...], be_ref[...])
    out = jnp.dot(t, w_out_ref[...], preferred_element_type=jnp.float32)
    out_ref[...] = out + b_out_ref[...][None, :]


_TCB = 1024
_GRID = (N_PAD // _TCB,)


def _rowblk(width):
    return pl.BlockSpec((_TCB, width), lambda i: (i, 0))


def _degblk():
    return pl.BlockSpec((2, _TCB, DEGW), lambda i: (0, i, 0))


def _pblk():
    return pl.BlockSpec((2, _TCB, H), lambda i: (0, i, 0))


def _full2(a, b):
    return pl.BlockSpec((a, b), lambda i: (0, 0))


def _full1(a):
    return pl.BlockSpec((a,), lambda i: (0,))


def _tc_first(x_pad, degp, w_in, b_in, w1):
    return pl.pallas_call(
        _tc_first_body,
        grid=_GRID,
        in_specs=[_rowblk(128), _degblk(), _full2(128, H), _full1(H),
                  _full2(H, H)],
        out_specs=_rowblk(H),
        out_shape=jax.ShapeDtypeStruct((N_PAD, H), jnp.float32),
    )(x_pad, degp, w_in, b_in, w1)


def _tc_mid(p, u, degp, b, g, be, w_next):
    return pl.pallas_call(
        _tc_mid_body,
        grid=_GRID,
        in_specs=[_pblk(), _rowblk(H), _degblk(), _full1(H), _full1(H),
                  _full1(H), _full2(H, H)],
        out_specs=_rowblk(H),
        out_shape=jax.ShapeDtypeStruct((N_PAD, H), jnp.float32),
    )(p, u, degp, b, g, be, w_next)


def _tc_last(p, u, degp, b, g, be, w_out, b_out):
    return pl.pallas_call(
        _tc_last_body,
        grid=_GRID,
        in_specs=[_pblk(), _rowblk(H), _degblk(), _full1(H), _full1(H),
                  _full1(H), _full2(H, 40), _full1(40)],
        out_specs=_rowblk(40),
        out_shape=jax.ShapeDtypeStruct((N_PAD, 40), jnp.float32),
    )(p, u, degp, b, g, be, w_out, b_out)


def kernel(x, edge_index, batch, W_in, b_in, W1, b1, g1, be1, W2, b2, g2,
           be2, W3, b3, g3, be3, W_out, b_out):
    x_pad = jnp.zeros((N_PAD, 128), jnp.float32).at[:N_NODES].set(x)
    pad_idx = (N_NODES + jnp.arange(E_PAD - E_EDGES, dtype=jnp.int32)
               % (N_PAD - N_NODES))
    src = jnp.concatenate([edge_index[0], pad_idx]).reshape(NWORK, NGRP, GRP)
    dst = jnp.concatenate([edge_index[1], pad_idx]).reshape(NWORK, NGRP, GRP)

    zeros_h = jnp.zeros((N_PAD, H), jnp.float32)
    zeros_d = jnp.zeros((N_PAD, DEGW), jnp.float32)
    ones_d = jnp.ones((GRP, DEGW), jnp.float32)

    degp = _deg_kernel(dst, ones_d, zeros_d)
    u1 = _tc_first(x_pad, degp, W_in, b_in, W1)
    p1 = _agg_kernel(u1, src, dst, zeros_h)
    u2 = _tc_mid(p1, u1, degp, b1, g1, be1, W2)
    p2 = _agg_kernel(u2, src, dst, zeros_h)
    u3 = _tc_mid(p2, u2, degp, b2, g2, be2, W3)
    p3 = _agg_kernel(u3, src, dst, zeros_h)
    out = _tc_last(p3, u3, degp, b3, g3, be3, W_out, b_out)
    return out[:N_NODES]


# speedup vs baseline: 2.5477x; 1.3977x over previous
import functools

import jax
import jax.numpy as jnp
from jax import lax
from jax.experimental import pallas as pl
from jax.experimental.pallas import tpu as pltpu
from jax.experimental.pallas import tpu_sc as plsc

N_NODES = 10000
N_PAD = 10240
E_EDGES = 320000
GRP = 128
NGRP = 79
NWORK = 32
E_PAD = NWORK * NGRP * GRP
ROWS_PER_TILE = N_PAD // 16
H = 64
DEGW = 16
GSZ = 4
NDMA = NGRP // GSZ

_sc_mesh = plsc.VectorSubcoreMesh(core_axis_name="c", subcore_axis_name="s")
_sc_params = pltpu.CompilerParams(use_tc_tiling_on_sc=False)


def _deg_body(dst_hbm, ones_hbm, zero_hbm, out_hbm, idx_v, ones_v, acc):
    c = lax.axis_index("c")
    s = lax.axis_index("s")
    wid = c * 16 + s
    rows = pl.ds(s * ROWS_PER_TILE, ROWS_PER_TILE)
    pltpu.sync_copy(zero_hbm.at[rows], acc.at[rows])
    pltpu.sync_copy(ones_hbm, ones_v)
    pltpu.sync_copy(dst_hbm.at[wid], idx_v)
    plsc.subcore_barrier()

    def body(j, carry):
        pltpu.sync_copy(ones_v, acc.at[idx_v.at[j]], add=True)
        return carry

    lax.fori_loop(0, NGRP, body, 0)
    plsc.subcore_barrier()
    pltpu.sync_copy(acc.at[rows], out_hbm.at[c, rows])


_deg_kernel = functools.partial(
    pl.kernel,
    out_type=jax.ShapeDtypeStruct((2, N_PAD, DEGW), jnp.float32),
    mesh=_sc_mesh,
    scratch_types=[
        pltpu.VMEM((NGRP, GRP), jnp.int32),
        pltpu.VMEM((GRP, DEGW), jnp.float32),
        pltpu.VMEM_SHARED((N_PAD, DEGW), jnp.float32),
    ],
    compiler_params=_sc_params,
)(_deg_body)


def _agg_body(u_hbm, src_hbm, dst_hbm, zero_hbm, out_hbm,
              idx_s, idx_d, rows_v, acc, gsem):
    c = lax.axis_index("c")
    s = lax.axis_index("s")
    wid = c * 16 + s
    rows = pl.ds(s * ROWS_PER_TILE, ROWS_PER_TILE)
    pltpu.sync_copy(zero_hbm.at[rows], acc.at[rows])
    pltpu.sync_copy(src_hbm.at[wid], idx_s)
    pltpu.sync_copy(dst_hbm.at[wid], idx_d)
    plsc.subcore_barrier()

    def fire(g, b):
        return pltpu.async_copy(u_hbm.at[idx_s.at[g]], rows_v.at[b], gsem)

    def scat(g, b):
        pltpu.sync_copy(rows_v.at[b], acc.at[idx_d.at[g]], add=True)

    d0 = fire(0, 0)

    def body(gg, carry):
        g = 2 * gg
        d1 = fire(g + 1, 1)
        d0.wait()
        scat(g, 0)
        d2 = fire(g + 2, 0)
        d1.wait()
        scat(g + 1, 1)
        return carry

    lax.fori_loop(0, (NGRP - 1) // 2, body, 0)
    d0.wait()
    scat(NGRP - 1, 0)

    plsc.subcore_barrier()
    pltpu.sync_copy(acc.at[rows], out_hbm.at[c, rows])


_agg_kernel = functools.partial(
    pl.kernel,
    out_type=jax.ShapeDtypeStruct((2, N_PAD, H), jnp.float32),
    mesh=_sc_mesh,
    scratch_types=[
        pltpu.VMEM((NGRP, GRP), jnp.int32),
        pltpu.VMEM((NGRP, GRP), jnp.int32),
        pltpu.VMEM((2, GRP, H), jnp.float32),
        pltpu.VMEM_SHARED((N_PAD, H), jnp.float32),
        pltpu.SemaphoreType.DMA,
    ],
    compiler_params=_sc_params,
)(_agg_body)


def _dinv_of(degp_blk):
    deg = degp_blk[0, :, 0] + degp_blk[1, :, 0] + 1.0
    return lax.rsqrt(jnp.maximum(deg, 1.0))


def _tc_first_body(x_ref, degp_ref, w_in_ref, b_in_ref, w1_ref, u_ref):
    dinv = _dinv_of(degp_ref[...])
    h = jnp.dot(x_ref[...], w_in_ref[...], preferred_element_type=jnp.float32)
    h = jax.nn.relu(h + b_in_ref[...][None, :])
    u = jnp.dot(h, w1_ref[...], preferred_element_type=jnp.float32)
    u_ref[...] = u * dinv[:, None]


def _layer_norm(t, g, b, eps=1e-5):
    mu = jnp.mean(t, axis=-1, keepdims=True)
    var = jnp.mean((t - mu) ** 2, axis=-1, keepdims=True)
    return (t - mu) * lax.rsqrt(var + eps) * g[None, :] + b[None, :]


def _tc_mid_body(p_ref, u_ref, degp_ref, b_ref, g_ref, be_ref, w_ref,
                 un_ref):
    dinv = _dinv_of(degp_ref[...])
    p = p_ref[...]
    agg = p[0] + p[1] + u_ref[...]
    t = agg * dinv[:, None] + b_ref[...][None, :]
    t = _layer_norm(jax.nn.relu(t), g_ref[...], be_ref[...])
    un = jnp.dot(t, w_ref[...], preferred_element_type=jnp.float32)
    un_ref[...] = un * dinv[:, None]


def _tc_last_body(p_ref, u_ref, degp_ref, b_ref, g_ref, be_ref, w_out_ref,
                  b_out_ref, out_ref):
    dinv = _dinv_of(degp_ref[...])
    p = p_ref[...]
    agg = p[0] + p[1] + u_ref[...]
    t = agg * dinv[:, None] + b_ref[...][None, :]
    t = _layer_norm(jax.nn.relu(t), g_ref[...], be_ref[...])
    out = jnp.dot(t, w_out_ref[...], preferred_element_type=jnp.float32)
    out_ref[...] = out + b_out_ref[...][None, :]


_TCB = 1024
_GRID = (N_PAD // _TCB,)


def _rowblk(width):
    return pl.BlockSpec((_TCB, width), lambda i: (i, 0))


def _degblk():
    return pl.BlockSpec((2, _TCB, DEGW), lambda i: (0, i, 0))


def _pblk():
    return pl.BlockSpec((2, _TCB, H), lambda i: (0, i, 0))


def _full2(a, b):
    return pl.BlockSpec((a, b), lambda i: (0, 0))


def _full1(a):
    return pl.BlockSpec((a,), lambda i: (0,))


def _tc_first(x_pad, degp, w_in, b_in, w1):
    return pl.pallas_call(
        _tc_first_body,
        grid=_GRID,
        in_specs=[_rowblk(128), _degblk(), _full2(128, H), _full1(H),
                  _full2(H, H)],
        out_specs=_rowblk(H),
        out_shape=jax.ShapeDtypeStruct((N_PAD, H), jnp.float32),
    )(x_pad, degp, w_in, b_in, w1)


def _tc_mid(p, u, degp, b, g, be, w_next):
    return pl.pallas_call(
        _tc_mid_body,
        grid=_GRID,
        in_specs=[_pblk(), _rowblk(H), _degblk(), _full1(H), _full1(H),
                  _full1(H), _full2(H, H)],
        out_specs=_rowblk(H),
        out_shape=jax.ShapeDtypeStruct((N_PAD, H), jnp.float32),
    )(p, u, degp, b, g, be, w_next)


def _tc_last(p, u, degp, b, g, be, w_out, b_out):
    return pl.pallas_call(
        _tc_last_body,
        grid=_GRID,
        in_specs=[_pblk(), _rowblk(H), _degblk(), _full1(H), _full1(H),
                  _full1(H), _full2(H, 40), _full1(40)],
        out_specs=_rowblk(40),
        out_shape=jax.ShapeDtypeStruct((N_PAD, 40), jnp.float32),
    )(p, u, degp, b, g, be, w_out, b_out)


def kernel(x, edge_index, batch, W_in, b_in, W1, b1, g1, be1, W2, b2, g2,
           be2, W3, b3, g3, be3, W_out, b_out):
    x_pad = jnp.zeros((N_PAD, 128), jnp.float32).at[:N_NODES].set(x)
    pad_idx = (N_NODES + jnp.arange(E_PAD - E_EDGES, dtype=jnp.int32)
               % (N_PAD - N_NODES))
    src = jnp.concatenate([edge_index[0], pad_idx]).reshape(NWORK, NGRP, GRP)
    dst = jnp.concatenate([edge_index[1], pad_idx]).reshape(NWORK, NGRP, GRP)

    zeros_h = jnp.zeros((N_PAD, H), jnp.float32)
    zeros_d = jnp.zeros((N_PAD, DEGW), jnp.float32)
    ones_d = jnp.ones((GRP, DEGW), jnp.float32)

    degp = _deg_kernel(dst, ones_d, zeros_d)
    u1 = _tc_first(x_pad, degp, W_in, b_in, W1)
    p1 = _agg_kernel(u1, src, dst, zeros_h)
    u2 = _tc_mid(p1, u1, degp, b1, g1, be1, W2)
    p2 = _agg_kernel(u2, src, dst, zeros_h)
    u3 = _tc_mid(p2, u2, degp, b2, g2, be2, W3)
    p3 = _agg_kernel(u3, src, dst, zeros_h)
    out = _tc_last(p3, u3, degp, b3, g3, be3, W_out, b_out)
    return out[:N_NODES]


# speedup vs baseline: 2.9038x; 1.1398x over previous
import functools

import jax
import jax.numpy as jnp
from jax import lax
from jax.experimental import pallas as pl
from jax.experimental.pallas import tpu as pltpu
from jax.experimental.pallas import tpu_sc as plsc

N_NODES = 10000
N_PAD = 10240
E_EDGES = 320000
GRP = 128
NGRP = 79
NWORK = 32
E_PAD = NWORK * NGRP * GRP
ROWS_PER_TILE = N_PAD // 16
H = 64
DEGW = 16
GSZ = 4
NDMA = NGRP // GSZ

_sc_mesh = plsc.VectorSubcoreMesh(core_axis_name="c", subcore_axis_name="s")
_sc_params = pltpu.CompilerParams(use_tc_tiling_on_sc=False)


def _deg_body(dst_hbm, ones_hbm, zero_hbm, out_hbm, idx_v, ones_v, acc, dsem):
    c = lax.axis_index("c")
    s = lax.axis_index("s")
    wid = c * 16 + s
    rows = pl.ds(s * ROWS_PER_TILE, ROWS_PER_TILE)
    pltpu.sync_copy(zero_hbm.at[rows], acc.at[rows])
    pltpu.sync_copy(ones_hbm, ones_v)
    pltpu.sync_copy(dst_hbm.at[wid], idx_v)
    plsc.subcore_barrier()

    dt = pltpu.make_async_copy(ones_v, acc.at[idx_v.at[0]], dsem)

    def df(j):
        pltpu.async_copy(ones_v, acc.at[idx_v.at[j]], dsem, add=True)

    df(0)
    df(1)
    df(2)
    df(3)

    def body(j, carry):
        dt.wait()
        df(j + 4)
        return carry

    lax.fori_loop(0, NGRP - 4, body, 0)
    dt.wait()
    dt.wait()
    dt.wait()
    dt.wait()
    plsc.subcore_barrier()
    pltpu.sync_copy(acc.at[rows], out_hbm.at[c, rows])


_deg_kernel = functools.partial(
    pl.kernel,
    out_type=jax.ShapeDtypeStruct((2, N_PAD, DEGW), jnp.float32),
    mesh=_sc_mesh,
    scratch_types=[
        pltpu.VMEM((NGRP, GRP), jnp.int32),
        pltpu.VMEM((GRP, DEGW), jnp.float32),
        pltpu.VMEM_SHARED((N_PAD, DEGW), jnp.float32),
        pltpu.SemaphoreType.DMA,
    ],
    compiler_params=_sc_params,
)(_deg_body)


def _agg_body(u_hbm, src_hbm, dst_hbm, zero_hbm, out_hbm,
              idx_s, idx_d, rows_v, acc, gsem, ssem):
    c = lax.axis_index("c")
    s = lax.axis_index("s")
    wid = c * 16 + s
    rows = pl.ds(s * ROWS_PER_TILE, ROWS_PER_TILE)
    pltpu.sync_copy(zero_hbm.at[rows], acc.at[rows])
    pltpu.sync_copy(src_hbm.at[wid], idx_s)
    pltpu.sync_copy(dst_hbm.at[wid], idx_d)
    plsc.subcore_barrier()

    gt = pltpu.make_async_copy(u_hbm.at[idx_s.at[0]], rows_v.at[0], gsem)
    st = pltpu.make_async_copy(rows_v.at[0], acc.at[idx_d.at[0]], ssem)

    def fg(g, b):
        pltpu.async_copy(u_hbm.at[idx_s.at[g]], rows_v.at[b], gsem)

    def sf(g, b):
        pltpu.async_copy(rows_v.at[b], acc.at[idx_d.at[g]], ssem, add=True)

    fg(0, 0)
    fg(1, 1)
    fg(2, 2)
    fg(3, 3)
    gt.wait()
    sf(0, 0)
    gt.wait()
    sf(1, 1)
    st.wait()
    st.wait()
    fg(4, 0)
    fg(5, 1)
    gt.wait()
    sf(2, 2)
    gt.wait()
    sf(3, 3)

    def body(pp, carry):
        g0 = 4 * pp
        st.wait()
        st.wait()
        fg(g0 + 2, 2)
        fg(g0 + 3, 3)
        gt.wait()
        sf(g0, 0)
        gt.wait()
        sf(g0 + 1, 1)
        st.wait()
        st.wait()
        fg(g0 + 4, 0)
        fg(g0 + 5, 1)
        gt.wait()
        sf(g0 + 2, 2)
        gt.wait()
        sf(g0 + 3, 3)
        return carry

    lax.fori_loop(1, 19, body, 0)
    st.wait()
    st.wait()
    fg(78, 2)
    gt.wait()
    sf(76, 0)
    gt.wait()
    sf(77, 1)
    st.wait()
    st.wait()
    gt.wait()
    sf(78, 2)
    st.wait()

    plsc.subcore_barrier()
    pltpu.sync_copy(acc.at[rows], out_hbm.at[c, rows])


_agg_kernel = functools.partial(
    pl.kernel,
    out_type=jax.ShapeDtypeStruct((2, N_PAD, H), jnp.float32),
    mesh=_sc_mesh,
    scratch_types=[
        pltpu.VMEM((NGRP, GRP), jnp.int32),
        pltpu.VMEM((NGRP, GRP), jnp.int32),
        pltpu.VMEM((4, GRP, H), jnp.float32),
        pltpu.VMEM_SHARED((N_PAD, H), jnp.float32),
        pltpu.SemaphoreType.DMA,
        pltpu.SemaphoreType.DMA,
    ],
    compiler_params=_sc_params,
)(_agg_body)


def _dinv_of(degp_blk):
    deg = degp_blk[0, :, 0] + degp_blk[1, :, 0] + 1.0
    return lax.rsqrt(jnp.maximum(deg, 1.0))


def _tc_first_body(x_ref, degp_ref, w_in_ref, b_in_ref, w1_ref, u_ref):
    dinv = _dinv_of(degp_ref[...])
    h = jnp.dot(x_ref[...], w_in_ref[...], preferred_element_type=jnp.float32)
    h = jax.nn.relu(h + b_in_ref[...][None, :])
    u = jnp.dot(h, w1_ref[...], preferred_element_type=jnp.float32)
    u_ref[...] = u * dinv[:, None]


def _layer_norm(t, g, b, eps=1e-5):
    mu = jnp.mean(t, axis=-1, keepdims=True)
    var = jnp.mean((t - mu) ** 2, axis=-1, keepdims=True)
    return (t - mu) * lax.rsqrt(var + eps) * g[None, :] + b[None, :]


def _tc_mid_body(p_ref, u_ref, degp_ref, b_ref, g_ref, be_ref, w_ref,
                 un_ref):
    dinv = _dinv_of(degp_ref[...])
    p = p_ref[...]
    agg = p[0] + p[1] + u_ref[...]
    t = agg * dinv[:, None] + b_ref[...][None, :]
    t = _layer_norm(jax.nn.relu(t), g_ref[...], be_ref[...])
    un = jnp.dot(t, w_ref[...], preferred_element_type=jnp.float32)
    un_ref[...] = un * dinv[:, None]


def _tc_last_body(p_ref, u_ref, degp_ref, b_ref, g_ref, be_ref, w_out_ref,
                  b_out_ref, out_ref):
    dinv = _dinv_of(degp_ref[...])
    p = p_ref[...]
    agg = p[0] + p[1] + u_ref[...]
    t = agg * dinv[:, None] + b_ref[...][None, :]
    t = _layer_norm(jax.nn.relu(t), g_ref[...], be_ref[...])
    out = jnp.dot(t, w_out_ref[...], preferred_element_type=jnp.float32)
    out_ref[...] = out + b_out_ref[...][None, :]


_TCB = 1024
_GRID = (N_PAD // _TCB,)


def _rowblk(width):
    return pl.BlockSpec((_TCB, width), lambda i: (i, 0))


def _degblk():
    return pl.BlockSpec((2, _TCB, DEGW), lambda i: (0, i, 0))


def _pblk():
    return pl.BlockSpec((2, _TCB, H), lambda i: (0, i, 0))


def _full2(a, b):
    return pl.BlockSpec((a, b), lambda i: (0, 0))


def _full1(a):
    return pl.BlockSpec((a,), lambda i: (0,))


def _tc_first(x_pad, degp, w_in, b_in, w1):
    return pl.pallas_call(
        _tc_first_body,
        grid=_GRID,
        in_specs=[_rowblk(128), _degblk(), _full2(128, H), _full1(H),
                  _full2(H, H)],
        out_specs=_rowblk(H),
        out_shape=jax.ShapeDtypeStruct((N_PAD, H), jnp.float32),
    )(x_pad, degp, w_in, b_in, w1)


def _tc_mid(p, u, degp, b, g, be, w_next):
    return pl.pallas_call(
        _tc_mid_body,
        grid=_GRID,
        in_specs=[_pblk(), _rowblk(H), _degblk(), _full1(H), _full1(H),
                  _full1(H), _full2(H, H)],
        out_specs=_rowblk(H),
        out_shape=jax.ShapeDtypeStruct((N_PAD, H), jnp.float32),
    )(p, u, degp, b, g, be, w_next)


def _tc_last(p, u, degp, b, g, be, w_out, b_out):
    return pl.pallas_call(
        _tc_last_body,
        grid=_GRID,
        in_specs=[_pblk(), _rowblk(H), _degblk(), _full1(H), _full1(H),
                  _full1(H), _full2(H, 40), _full1(40)],
        out_specs=_rowblk(40),
        out_shape=jax.ShapeDtypeStruct((N_PAD, 40), jnp.float32),
    )(p, u, degp, b, g, be, w_out, b_out)


def kernel(x, edge_index, batch, W_in, b_in, W1, b1, g1, be1, W2, b2, g2,
           be2, W3, b3, g3, be3, W_out, b_out):
    x_pad = jnp.zeros((N_PAD, 128), jnp.float32).at[:N_NODES].set(x)
    pad_idx = (N_NODES + jnp.arange(E_PAD - E_EDGES, dtype=jnp.int32)
               % (N_PAD - N_NODES))
    src = jnp.concatenate([edge_index[0], pad_idx]).reshape(NWORK, NGRP, GRP)
    dst = jnp.concatenate([edge_index[1], pad_idx]).reshape(NWORK, NGRP, GRP)

    zeros_h = jnp.zeros((N_PAD, H), jnp.float32)
    zeros_d = jnp.zeros((N_PAD, DEGW), jnp.float32)
    ones_d = jnp.ones((GRP, DEGW), jnp.float32)

    degp = _deg_kernel(dst, ones_d, zeros_d)
    u1 = _tc_first(x_pad, degp, W_in, b_in, W1)
    p1 = _agg_kernel(u1, src, dst, zeros_h)
    u2 = _tc_mid(p1, u1, degp, b1, g1, be1, W2)
    p2 = _agg_kernel(u2, src, dst, zeros_h)
    u3 = _tc_mid(p2, u2, degp, b2, g2, be2, W3)
    p3 = _agg_kernel(u3, src, dst, zeros_h)
    out = _tc_last(p3, u3, degp, b3, g3, be3, W_out, b_out)
    return out[:N_NODES]
